# baseline (speedup 1.0000x reference)
import functools

import jax
import jax.numpy as jnp
import numpy as np
from jax import lax
from jax.experimental import pallas as pl
from jax.experimental.pallas import tpu as pltpu
from jax.experimental.pallas import tpu_sc as plsc

NC = 2
NS = 16
L = 16
NW = NC * NS

P = 48
H = 512
W = 512
NBUF = 4
RB = 8
BPP = H // (NW * RB)
STEPS = P * BPP
SEG = RB * W
NLUT = 256
VITER = SEG // L
CPR = W // L

_MAGIC = np.float32(2.0 ** 23)


def _mesh():
    return plsc.VectorSubcoreMesh(
        core_axis_name="c", subcore_axis_name="s", num_cores=NC, num_subcores=NS
    )


def _sc_body(img_h, t1_h, t2_h, o1_h, o2_h,
             t1_v, t2_v, in_v, o1_v, o2_v, *sems):
    sem_in = sems[:NBUF]
    sem_out = sems[NBUF:]
    wid = lax.axis_index("s") * NC + lax.axis_index("c")
    pltpu.async_copy(t1_h, t1_v, sem_in[0]).wait()
    pltpu.async_copy(t2_h, t2_v, sem_in[0]).wait()

    def rows(step):
        p = step // BPP
        r0 = pl.multiple_of((wid * BPP + step % BPP) * RB, RB)
        return p, r0

    def issue_in(step, k):
        p, r0 = rows(step)
        pltpu.async_copy(img_h.at[p, pl.ds(r0, RB)], in_v.at[k], sem_in[k])

    def wait_in(step, k):
        p, r0 = rows(step)
        pltpu.make_async_copy(img_h.at[p, pl.ds(r0, RB)], in_v.at[k],
                              sem_in[k]).wait()

    def issue_out(step, k):
        p, r0 = rows(step)
        pltpu.async_copy(o1_v.at[k], o1_h.at[p, pl.ds(r0, RB)], sem_out[k])
        pltpu.async_copy(o2_v.at[k], o2_h.at[p, pl.ds(r0, RB)], sem_out[k])

    def wait_out(step, k):
        p, r0 = rows(step)
        pltpu.make_async_copy(o1_v.at[k], o1_h.at[p, pl.ds(r0, RB)],
                              sem_out[k]).wait()
        pltpu.make_async_copy(o2_v.at[k], o2_h.at[p, pl.ds(r0, RB)],
                              sem_out[k]).wait()

    def compute(step, k):
        p = step // BPP
        magic = (p * NLUT).astype(jnp.float32) + _MAGIC
        magic_v = jnp.zeros((L,), jnp.float32) + magic

        @plsc.parallel_loop(0, VITER, 1, unroll=8)
        def _(i):
            r = i // CPR
            c = (i % CPR) * L
            x = in_v[k, r, pl.ds(c, L)]
            f = x * jnp.float32(255.0) + magic_v
            idx = lax.bitcast_convert_type(f, jnp.int32) & jnp.int32(0x3FFF)
            o1_v[k, r, pl.ds(c, L)] = plsc.load_gather(t1_v, [idx])
            o2_v[k, r, pl.ds(c, L)] = plsc.load_gather(t2_v, [idx])

    for s in range(NBUF - 1):
        issue_in(s, s)

    def ring(g, _):
        for par in range(NBUF):
            step = NBUF * g + par
            wait_in(step, par)
            nstep = step + NBUF - 1
            kpre = (par + NBUF - 1) % NBUF

            @pl.when(nstep < STEPS)
            def _():
                issue_in(nstep, kpre)

            @pl.when(g > 0)
            def _():
                wait_out(step - NBUF, par)

            compute(step, par)
            issue_out(step, par)
        return 0

    lax.fori_loop(0, STEPS // NBUF, ring, 0)
    for s in range(NBUF):
        wait_out(STEPS - NBUF + s, s)


def _tc_body(t3_ref, img_ref, out_ref):
    x = img_ref[0]
    tab = t3_ref[0, 0]
    tlo = jnp.broadcast_to(tab[:128].reshape(1, 128), (H, 128))
    thi = jnp.broadcast_to(tab[128:].reshape(1, 128), (H, 128))
    f = x * jnp.float32(255.0) + _MAGIC
    w = lax.bitcast_convert_type(f, jnp.int32)
    lo = w & jnp.int32(0x7F)
    hi = (w & jnp.int32(0x80)) != 0
    glo = jnp.take_along_axis(tlo, lo, axis=1)
    ghi = jnp.take_along_axis(thi, lo, axis=1)
    out_ref[0] = jnp.where(hi, ghi, glo)


def _tc_lut(img3, t3):
    return pl.pallas_call(
        _tc_body,
        grid=(P,),
        in_specs=[
            pl.BlockSpec((1, 1, NLUT), lambda p: (p, 0, 0)),
            pl.BlockSpec((1, H, W), lambda p: (p, 0, 0)),
        ],
        out_specs=pl.BlockSpec((1, H, W), lambda p: (p, 0, 0)),
        out_shape=jax.ShapeDtypeStruct((P, H, W), jnp.float32),
    )(t3, img3)


@functools.partial(jax.jit)
def _run(img3, t1, t2, t3):
    out_t = tuple(jax.ShapeDtypeStruct((P, H, W), jnp.float32) for _ in range(2))
    scratch = [
        pltpu.VMEM((P * NLUT,), jnp.float32),
        pltpu.VMEM((P * NLUT,), jnp.float32),
        pltpu.VMEM((NBUF, RB, W), jnp.float32),
        pltpu.VMEM((NBUF, RB, W), jnp.float32),
        pltpu.VMEM((NBUF, RB, W), jnp.float32),
    ] + [pltpu.SemaphoreType.DMA] * (2 * NBUF)
    f = pl.kernel(
        _sc_body, out_type=out_t, mesh=_mesh(), scratch_types=scratch,
        compiler_params=pltpu.CompilerParams(
            needs_layout_passes=False, use_tc_tiling_on_sc=True,
        ),
    )
    o1, o2 = f(img3, t1, t2)
    o3 = _tc_lut(img3, t3)
    return o1, o2, o3


def kernel(img, tf1, tf2, tf3):
    B, C, _, _ = img.shape
    o1, o2, o3 = _run(
        img.reshape(P, H, W),
        tf1.reshape(P * NLUT),
        tf2.reshape(P * NLUT),
        tf3.reshape(P, 1, NLUT),
    )
    shp = (B, C, H, W)
    return (o1.reshape(shp), o2.reshape(shp), o3.reshape(shp))


# speedup vs baseline: 1.1742x; 1.1742x over previous
import functools

import jax
import jax.numpy as jnp
import numpy as np
from jax import lax
from jax.experimental import pallas as pl
from jax.experimental.pallas import tpu as pltpu
from jax.experimental.pallas import tpu_sc as plsc

NC = 2
NS = 16
L = 16
NW = NC * NS

P = 48
H = 512
W = 512
NBUF = 4
RB = 8
BPP = H // (NW * RB)
STEPS = P * BPP
SEG = RB * W
NLUT = 256
VITER = SEG // L
CPR = W // L

_MAGIC = np.float32(2.0 ** 23)


def _mesh():
    return plsc.VectorSubcoreMesh(
        core_axis_name="c", subcore_axis_name="s", num_cores=NC, num_subcores=NS
    )


def _body(img_h, t1_h, t2_h, t3_h, o1_h, o2_h, o3_h,
          t1_v, t2_v, t3_v, in_v, o1_v, o2_v, o3_v, *sems):
    sem_in = sems[:NBUF]
    sem_out = sems[NBUF:]
    wid = lax.axis_index("s") * NC + lax.axis_index("c")

    def rows(step):
        p = step // BPP
        r0 = pl.multiple_of((wid * BPP + step % BPP) * RB, RB)
        return p, r0

    def issue_in(step, k):
        p, r0 = rows(step)
        pltpu.async_copy(img_h.at[p, pl.ds(r0, RB)], in_v.at[k], sem_in[k])

    def wait_in(step, k):
        p, r0 = rows(step)
        pltpu.make_async_copy(img_h.at[p, pl.ds(r0, RB)], in_v.at[k],
                              sem_in[k]).wait()

    def issue_out(step, k):
        p, r0 = rows(step)
        pltpu.async_copy(o1_v.at[k], o1_h.at[p, pl.ds(r0, RB)], sem_out[k])
        pltpu.async_copy(o2_v.at[k], o2_h.at[p, pl.ds(r0, RB)], sem_out[k])
        pltpu.async_copy(o3_v.at[k], o3_h.at[p, pl.ds(r0, RB)], sem_out[k])

    def wait_out(step, k):
        p, r0 = rows(step)
        pltpu.make_async_copy(o1_v.at[k], o1_h.at[p, pl.ds(r0, RB)],
                              sem_out[k]).wait()
        pltpu.make_async_copy(o2_v.at[k], o2_h.at[p, pl.ds(r0, RB)],
                              sem_out[k]).wait()
        pltpu.make_async_copy(o3_v.at[k], o3_h.at[p, pl.ds(r0, RB)],
                              sem_out[k]).wait()

    def compute(step, k):
        p = step // BPP
        magic = (p * NLUT).astype(jnp.float32) + _MAGIC
        magic_v = jnp.zeros((L,), jnp.float32) + magic

        @plsc.parallel_loop(0, VITER, 1, unroll=8)
        def _(i):
            r = i // CPR
            c = (i % CPR) * L
            x = in_v[k, r, pl.ds(c, L)]
            f = x * jnp.float32(255.0) + magic_v
            idx = lax.bitcast_convert_type(f, jnp.int32) & jnp.int32(0x3FFF)
            o1_v[k, r, pl.ds(c, L)] = plsc.load_gather(t1_v, [idx])
            o2_v[k, r, pl.ds(c, L)] = plsc.load_gather(t2_v, [idx])
            o3_v[k, r, pl.ds(c, L)] = plsc.load_gather(t3_v, [idx])

    for s in range(NBUF - 1):
        issue_in(s, s)

    lut_sem = sem_out[0]
    pltpu.async_copy(t1_h, t1_v, lut_sem)
    pltpu.async_copy(t2_h, t2_v, lut_sem)
    pltpu.async_copy(t3_h, t3_v, lut_sem)
    pltpu.make_async_copy(t1_h, t1_v, lut_sem).wait()
    pltpu.make_async_copy(t2_h, t2_v, lut_sem).wait()
    pltpu.make_async_copy(t3_h, t3_v, lut_sem).wait()

    def ring(g, _):
        for par in range(NBUF):
            step = NBUF * g + par
            wait_in(step, par)
            nstep = step + NBUF - 1
            kpre = (par + NBUF - 1) % NBUF

            @pl.when(nstep < STEPS)
            def _():
                issue_in(nstep, kpre)

            @pl.when(g > 0)
            def _():
                wait_out(step - NBUF, par)

            compute(step, par)
            issue_out(step, par)
        return 0

    lax.fori_loop(0, STEPS // NBUF, ring, 0)
    for s in range(NBUF):
        wait_out(STEPS - NBUF + s, s)


@functools.partial(jax.jit)
def _run(img3, t1, t2, t3):
    out_t = tuple(jax.ShapeDtypeStruct((P, H, W), jnp.float32) for _ in range(3))
    scratch = [
        pltpu.VMEM((P * NLUT,), jnp.float32),
        pltpu.VMEM((P * NLUT,), jnp.float32),
        pltpu.VMEM((P * NLUT,), jnp.float32),
        pltpu.VMEM((NBUF, RB, W), jnp.float32),
        pltpu.VMEM((NBUF, RB, W), jnp.float32),
        pltpu.VMEM((NBUF, RB, W), jnp.float32),
        pltpu.VMEM((NBUF, RB, W), jnp.float32),
    ] + [pltpu.SemaphoreType.DMA] * (2 * NBUF)
    f = pl.kernel(
        _body, out_type=out_t, mesh=_mesh(), scratch_types=scratch,
        compiler_params=pltpu.CompilerParams(
            needs_layout_passes=False, use_tc_tiling_on_sc=True,
        ),
    )
    return f(img3, t1, t2, t3)


def kernel(img, tf1, tf2, tf3):
    B, C, _, _ = img.shape
    o1, o2, o3 = _run(
        img.reshape(P, H, W),
        tf1.reshape(P * NLUT),
        tf2.reshape(P * NLUT),
        tf3.reshape(P * NLUT),
    )
    shp = (B, C, H, W)
    return (o1.reshape(shp), o2.reshape(shp), o3.reshape(shp))


# speedup vs baseline: 1.1877x; 1.0115x over previous
import functools

import jax
import jax.numpy as jnp
import numpy as np
from jax import lax
from jax.experimental import pallas as pl
from jax.experimental.pallas import tpu as pltpu
from jax.experimental.pallas import tpu_sc as plsc

NC = 2
NS = 16
L = 16
NW = NC * NS

P = 48
H = 512
W = 512
NBUF = 4
RB = 8
BPP = H // (NW * RB)
STEPS = P * BPP
SEG = RB * W
NLUT = 256
VITER = SEG // L
CPR = W // L

_MAGIC = np.float32(2.0 ** 23)


def _mesh():
    return plsc.VectorSubcoreMesh(
        core_axis_name="c", subcore_axis_name="s", num_cores=NC, num_subcores=NS
    )


def _body(img_h, tc_h, o1_h, o2_h, o3_h,
          t1_v, t2_v, t3_v, in_v, o1_v, o2_v, o3_v, *sems):
    sem_in = sems[:NBUF]
    sem_out = sems[NBUF:]
    wid = lax.axis_index("s") * NC + lax.axis_index("c")

    def rows(step):
        p = step // BPP
        r0 = pl.multiple_of((wid * BPP + step % BPP) * RB, RB)
        return p, r0

    def issue_in(step, k):
        p, r0 = rows(step)
        pltpu.async_copy(img_h.at[p, pl.ds(r0, RB)], in_v.at[k], sem_in[k])

    def wait_in(step, k):
        p, r0 = rows(step)
        pltpu.make_async_copy(img_h.at[p, pl.ds(r0, RB)], in_v.at[k],
                              sem_in[k]).wait()

    def issue_out(step, k):
        p, r0 = rows(step)
        pltpu.async_copy(o1_v.at[k], o1_h.at[p, pl.ds(r0, RB)], sem_out[k])
        pltpu.async_copy(o2_v.at[k], o2_h.at[p, pl.ds(r0, RB)], sem_out[k])
        pltpu.async_copy(o3_v.at[k], o3_h.at[p, pl.ds(r0, RB)], sem_out[k])

    def wait_out(step, k):
        p, r0 = rows(step)
        pltpu.make_async_copy(o1_v.at[k], o1_h.at[p, pl.ds(r0, RB)],
                              sem_out[k]).wait()
        pltpu.make_async_copy(o2_v.at[k], o2_h.at[p, pl.ds(r0, RB)],
                              sem_out[k]).wait()
        pltpu.make_async_copy(o3_v.at[k], o3_h.at[p, pl.ds(r0, RB)],
                              sem_out[k]).wait()

    def compute(step, k):
        p = step // BPP
        magic = (p * NLUT).astype(jnp.float32) + _MAGIC
        magic_v = jnp.zeros((L,), jnp.float32) + magic

        @plsc.parallel_loop(0, VITER, 1, unroll=8)
        def _(i):
            r = i // CPR
            c = (i % CPR) * L
            x = in_v[k, r, pl.ds(c, L)]
            f = x * jnp.float32(255.0) + magic_v
            idx = lax.bitcast_convert_type(f, jnp.int32) & jnp.int32(0x3FFF)
            o1_v[k, r, pl.ds(c, L)] = plsc.load_gather(t1_v, [idx])
            o2_v[k, r, pl.ds(c, L)] = plsc.load_gather(t2_v, [idx])
            o3_v[k, r, pl.ds(c, L)] = plsc.load_gather(t3_v, [idx])

    for s in range(NBUF - 1):
        issue_in(s, s)

    lut_sem = sem_out[0]
    sz = P * NLUT
    for j, t_v in enumerate((t1_v, t2_v, t3_v)):
        pltpu.async_copy(tc_h.at[pl.ds(j * sz, sz)], t_v, lut_sem)
    for j, t_v in enumerate((t1_v, t2_v, t3_v)):
        pltpu.make_async_copy(tc_h.at[pl.ds(j * sz, sz)], t_v, lut_sem).wait()

    def ring(g, _):
        for par in range(NBUF):
            step = NBUF * g + par
            wait_in(step, par)
            nstep = step + NBUF - 1
            kpre = (par + NBUF - 1) % NBUF

            @pl.when(nstep < STEPS)
            def _():
                issue_in(nstep, kpre)

            @pl.when(g > 0)
            def _():
                wait_out(step - NBUF, par)

            compute(step, par)
            issue_out(step, par)
        return 0

    lax.fori_loop(0, STEPS // NBUF, ring, 0)
    for s in range(NBUF):
        wait_out(STEPS - NBUF + s, s)


@functools.partial(jax.jit)
def _run(img3, tcat):
    out_t = tuple(jax.ShapeDtypeStruct((P, H, W), jnp.float32) for _ in range(3))
    scratch = [
        pltpu.VMEM((P * NLUT,), jnp.float32),
        pltpu.VMEM((P * NLUT,), jnp.float32),
        pltpu.VMEM((P * NLUT,), jnp.float32),
        pltpu.VMEM((NBUF, RB, W), jnp.float32),
        pltpu.VMEM((NBUF, RB, W), jnp.float32),
        pltpu.VMEM((NBUF, RB, W), jnp.float32),
        pltpu.VMEM((NBUF, RB, W), jnp.float32),
    ] + [pltpu.SemaphoreType.DMA] * (2 * NBUF)
    f = pl.kernel(
        _body, out_type=out_t, mesh=_mesh(), scratch_types=scratch,
        compiler_params=pltpu.CompilerParams(
            needs_layout_passes=False, use_tc_tiling_on_sc=True,
        ),
    )
    return f(img3, tcat)


def kernel(img, tf1, tf2, tf3):
    B, C, _, _ = img.shape
    tcat = jnp.concatenate(
        [tf1.reshape(P * NLUT), tf2.reshape(P * NLUT), tf3.reshape(P * NLUT)]
    )
    o1, o2, o3 = _run(img.reshape(P, H, W), tcat)
    shp = (B, C, H, W)
    return (o1.reshape(shp), o2.reshape(shp), o3.reshape(shp))
